# Initial kernel scaffold; baseline (speedup 1.0000x reference)
import functools

import jax
import jax.numpy as jnp
from jax import lax
from jax.experimental import pallas as pl
from jax.experimental.pallas import tpu as pltpu
from jax.experimental.pallas import tpu_sc as plsc

NN = 10000
NE = 320000
DF = 128
DH = 16
DC = 40

NCORES = 2
NSUB = 16
NW = NCORES * NSUB
EPT = NE // NW
CH = 80
NCHUNK = EPT // CH
ZR = NN // NSUB

_mesh = plsc.VectorSubcoreMesh(core_axis_name="c", subcore_axis_name="s")


@functools.partial(
    pl.kernel,
    out_type=jax.ShapeDtypeStruct((NW, NN), jnp.float32),
    mesh=_mesh,
    scratch_types=[
        pltpu.VMEM((EPT,), jnp.int32),
        pltpu.VMEM((EPT,), jnp.float32),
        pltpu.VMEM((NN,), jnp.float32),
    ],
)
def _sc_deg(col_hbm, ew_hbm, out_hbm, col_v, ew_v, acc_v):
    wid = lax.axis_index("s") * NCORES + lax.axis_index("c")
    base = wid * EPT
    pltpu.sync_copy(col_hbm.at[pl.ds(base, EPT)], col_v)
    pltpu.sync_copy(ew_hbm.at[pl.ds(base, EPT)], ew_v)
    zeros = jnp.zeros((16,), jnp.float32)

    def zbody(i, carry):
        acc_v[pl.ds(i * 16, 16)] = zeros
        return carry

    lax.fori_loop(0, NN // 16, zbody, 0, unroll=8)

    def body(i, carry):
        idx = col_v[pl.ds(i * 16, 16)]
        w = ew_v[pl.ds(i * 16, 16)]
        plsc.addupdate_scatter(acc_v, [idx], w)
        return carry

    lax.fori_loop(0, EPT // 16, body, 0, unroll=8)
    pltpu.sync_copy(acc_v, out_hbm.at[wid])


@functools.partial(
    pl.kernel,
    out_type=jax.ShapeDtypeStruct((NCORES, NN, DH), jnp.float32),
    mesh=_mesh,
    scratch_types=[
        pltpu.VMEM((NCHUNK, CH), jnp.int32),
        pltpu.VMEM((NCHUNK, CH), jnp.int32),
        pltpu.VMEM((NCHUNK, CH), jnp.float32),
        pltpu.VMEM((CH, DH), jnp.float32),
        pltpu.VMEM((ZR, DH), jnp.float32),
        pltpu.VMEM_SHARED((NN, DH), jnp.float32),
        pltpu.SemaphoreType.DMA,
    ],
)
def _sc_edge(h_hbm, row_hbm, col_hbm, ew_hbm, out_hbm,
             row_v, col_v, ew_v, msg_v, zero_v, acc_sh, gsem):
    cid = lax.axis_index("c")
    sid = lax.axis_index("s")
    wid = sid * NCORES + cid
    pltpu.sync_copy(row_hbm.at[wid], row_v)
    pltpu.sync_copy(col_hbm.at[wid], col_v)
    pltpu.sync_copy(ew_hbm.at[wid], ew_v)

    zeros = jnp.zeros((16,), jnp.float32)

    def zbody(r, carry):
        zero_v[r, :] = zeros
        return carry

    lax.fori_loop(0, ZR, zbody, 0, unroll=8)
    pltpu.sync_copy(zero_v, acc_sh.at[pl.ds(sid * ZR, ZR)])
    plsc.subcore_barrier()

    def chunk(c, carry):
        pltpu.async_copy(h_hbm.at[row_v.at[c]], msg_v, gsem).wait()

        def scale(e, c2):
            w = ew_v[c2, e]
            msg_v[e, :] = msg_v[e, :] * w
            return c2

        lax.fori_loop(0, CH, scale, c, unroll=8)
        pltpu.sync_copy(msg_v, acc_sh.at[col_v.at[c]], add=True)
        return carry

    lax.fori_loop(0, NCHUNK, chunk, 0)
    plsc.subcore_barrier()
    pltpu.sync_copy(acc_sh.at[pl.ds(sid * ZR, ZR)],
                    out_hbm.at[cid, pl.ds(sid * ZR, ZR)])


_BR = 2000


def _tc_pre_body(degt_ref, x_ref, w1_ref, h1_ref, h1s_ref, dis_ref):
    deg = jnp.sum(degt_ref[...], axis=1, keepdims=True) + 1.0
    dis = jnp.where(deg > 0, lax.rsqrt(jnp.maximum(deg, 1e-12)), 0.0)
    h1 = jnp.dot(x_ref[...], w1_ref[...], preferred_element_type=jnp.float32)
    dis16 = jnp.broadcast_to(dis, (_BR, DH))
    h1_ref[...] = h1
    dis_ref[...] = dis16
    h1s_ref[...] = dis16 * h1


def _tc_pre(degt, x, w1):
    grid = NN // _BR
    return pl.pallas_call(
        _tc_pre_body,
        grid=(grid,),
        in_specs=[
            pl.BlockSpec((_BR, NW), lambda i: (i, 0)),
            pl.BlockSpec((_BR, DF), lambda i: (i, 0)),
            pl.BlockSpec((DF, DH), lambda i: (0, 0)),
        ],
        out_specs=[
            pl.BlockSpec((_BR, DH), lambda i: (i, 0)),
            pl.BlockSpec((_BR, DH), lambda i: (i, 0)),
            pl.BlockSpec((_BR, DH), lambda i: (i, 0)),
        ],
        out_shape=[
            jax.ShapeDtypeStruct((NN, DH), jnp.float32),
            jax.ShapeDtypeStruct((NN, DH), jnp.float32),
            jax.ShapeDtypeStruct((NN, DH), jnp.float32),
        ],
    )(degt, x, w1)


def _tc_mid_body(p_ref, h1_ref, dis_ref, b1_ref, z1_ref, z1s_ref):
    dis = dis_ref[...]
    p = p_ref[0] + p_ref[1]
    z = dis * p + dis * dis * h1_ref[...] + b1_ref[...]
    z1 = jnp.maximum(z, 0.0)
    z1_ref[...] = z1
    z1s_ref[...] = dis * z1


def _tc_mid(p, h1, dis, b1):
    grid = NN // _BR
    return pl.pallas_call(
        _tc_mid_body,
        grid=(grid,),
        in_specs=[
            pl.BlockSpec((NCORES, _BR, DH), lambda i: (0, i, 0)),
            pl.BlockSpec((_BR, DH), lambda i: (i, 0)),
            pl.BlockSpec((_BR, DH), lambda i: (i, 0)),
            pl.BlockSpec((1, DH), lambda i: (0, 0)),
        ],
        out_specs=[
            pl.BlockSpec((_BR, DH), lambda i: (i, 0)),
            pl.BlockSpec((_BR, DH), lambda i: (i, 0)),
        ],
        out_shape=[
            jax.ShapeDtypeStruct((NN, DH), jnp.float32),
            jax.ShapeDtypeStruct((NN, DH), jnp.float32),
        ],
    )(p, h1, dis, b1)


def _tc_post_body(p_ref, z1_ref, dis_ref, w2_ref, b2_ref, out_ref):
    dis = dis_ref[...]
    p = p_ref[0] + p_ref[1]
    agg = dis * p + dis * dis * z1_ref[...]
    o = jnp.dot(agg, w2_ref[...], preferred_element_type=jnp.float32)
    o = o + b2_ref[...]
    m = jnp.max(o, axis=1, keepdims=True)
    lse = jnp.log(jnp.sum(jnp.exp(o - m), axis=1, keepdims=True)) + m
    out_ref[...] = o - lse


def _tc_post(p, z1, dis, w2, b2):
    grid = NN // _BR
    return pl.pallas_call(
        _tc_post_body,
        grid=(grid,),
        in_specs=[
            pl.BlockSpec((NCORES, _BR, DH), lambda i: (0, i, 0)),
            pl.BlockSpec((_BR, DH), lambda i: (i, 0)),
            pl.BlockSpec((_BR, DH), lambda i: (i, 0)),
            pl.BlockSpec((DH, DC), lambda i: (0, 0)),
            pl.BlockSpec((1, DC), lambda i: (0, 0)),
        ],
        out_specs=pl.BlockSpec((_BR, DC), lambda i: (i, 0)),
        out_shape=jax.ShapeDtypeStruct((NN, DC), jnp.float32),
    )(p, z1, dis, w2, b2)


def kernel(x, edge_index, edge_attr, W1, b1, W2, b2):
    row = edge_index[0].astype(jnp.int32)
    col = edge_index[1].astype(jnp.int32)
    ew = edge_attr.astype(jnp.float32)
    row3 = row.reshape(NW, NCHUNK, CH)
    col3 = col.reshape(NW, NCHUNK, CH)
    ew3 = ew.reshape(NW, NCHUNK, CH)

    deg_parts = _sc_deg(col, ew)
    degt = jnp.transpose(deg_parts)
    h1, h1s, dis = _tc_pre(degt, x, W1)
    p1 = _sc_edge(h1s, row3, col3, ew3)
    z1, z1s = _tc_mid(p1, h1, dis, b1.reshape(1, DH))
    p2 = _sc_edge(z1s, row3, col3, ew3)
    out = _tc_post(p2, z1, dis, W2, b2.reshape(1, DC))
    return out


# speedup vs baseline: 28.2752x; 28.2752x over previous
import functools

import jax
import jax.numpy as jnp
from jax import lax
from jax.experimental import pallas as pl
from jax.experimental.pallas import tpu as pltpu
from jax.experimental.pallas import tpu_sc as plsc

NN = 10000
NE = 320000
DF = 128
DH = 16
DC = 40

NCORES = 2
NSUB = 16
NW = NCORES * NSUB
EPT = NE // NW
CH = 80
NCHUNK = EPT // CH
ZR = NN // NSUB

_mesh = plsc.VectorSubcoreMesh(core_axis_name="c", subcore_axis_name="s")


@functools.partial(
    pl.kernel,
    out_type=jax.ShapeDtypeStruct((NW, NN), jnp.float32),
    mesh=_mesh,
    compiler_params=pltpu.CompilerParams(needs_layout_passes=False, use_tc_tiling_on_sc=False),
    scratch_types=[
        pltpu.VMEM((EPT,), jnp.int32),
        pltpu.VMEM((EPT,), jnp.float32),
        pltpu.VMEM((NN,), jnp.float32),
    ],
)
def _sc_deg(col_hbm, ew_hbm, out_hbm, col_v, ew_v, acc_v):
    wid = lax.axis_index("s") * NCORES + lax.axis_index("c")
    base = wid * EPT
    pltpu.sync_copy(col_hbm.at[pl.ds(base, EPT)], col_v)
    pltpu.sync_copy(ew_hbm.at[pl.ds(base, EPT)], ew_v)
    zeros = jnp.zeros((16,), jnp.float32)

    def zbody(i, carry):
        acc_v[pl.ds(i * 16, 16)] = zeros
        return carry

    lax.fori_loop(0, NN // 16, zbody, 0, unroll=8)

    def body(i, carry):
        idx = col_v[pl.ds(i * 16, 16)]
        w = ew_v[pl.ds(i * 16, 16)]
        plsc.addupdate_scatter(acc_v, [idx], w)
        return carry

    lax.fori_loop(0, EPT // 16, body, 0, unroll=8)
    pltpu.sync_copy(acc_v, out_hbm.at[wid])


@functools.partial(
    pl.kernel,
    out_type=jax.ShapeDtypeStruct((NCORES, NN, DH), jnp.float32),
    mesh=_mesh,
    compiler_params=pltpu.CompilerParams(needs_layout_passes=False, use_tc_tiling_on_sc=False),
    scratch_types=[
        pltpu.VMEM((NCHUNK, CH), jnp.int32),
        pltpu.VMEM((NCHUNK, CH), jnp.int32),
        pltpu.VMEM((NCHUNK, CH), jnp.float32),
        pltpu.VMEM((CH, DH), jnp.float32),
        pltpu.VMEM((ZR, DH), jnp.float32),
        pltpu.VMEM_SHARED((NN, DH), jnp.float32),
        pltpu.SemaphoreType.DMA,
    ],
)
def _sc_edge(h_hbm, row_hbm, col_hbm, ew_hbm, out_hbm,
             row_v, col_v, ew_v, msg_v, zero_v, acc_sh, gsem):
    cid = lax.axis_index("c")
    sid = lax.axis_index("s")
    wid = sid * NCORES + cid
    pltpu.sync_copy(row_hbm.at[wid], row_v)
    pltpu.sync_copy(col_hbm.at[wid], col_v)
    pltpu.sync_copy(ew_hbm.at[wid], ew_v)

    zeros = jnp.zeros((16,), jnp.float32)

    def zbody(r, carry):
        zero_v[r, :] = zeros
        return carry

    lax.fori_loop(0, ZR, zbody, 0, unroll=8)
    pltpu.sync_copy(zero_v, acc_sh.at[pl.ds(sid * ZR, ZR)])
    plsc.subcore_barrier()

    def chunk(c, carry):
        pltpu.async_copy(h_hbm.at[row_v.at[c]], msg_v, gsem).wait()
        for g in range(CH // 16):
            wvec = ew_v[c, pl.ds(g * 16, 16)]
            for e in range(16):
                r = g * 16 + e
                msg_v[r, :] = msg_v[r, :] * wvec[e]
        pltpu.sync_copy(msg_v, acc_sh.at[col_v.at[c]], add=True)
        return carry

    lax.fori_loop(0, NCHUNK, chunk, 0)
    plsc.subcore_barrier()
    pltpu.sync_copy(acc_sh.at[pl.ds(sid * ZR, ZR)],
                    out_hbm.at[cid, pl.ds(sid * ZR, ZR)])


_BR = 2000


def _tc_pre_body(degt_ref, x_ref, w1_ref, h1_ref, h1s_ref, dis_ref):
    deg = jnp.sum(degt_ref[...], axis=1, keepdims=True) + 1.0
    dis = jnp.where(deg > 0, lax.rsqrt(jnp.maximum(deg, 1e-12)), 0.0)
    h1 = jnp.dot(x_ref[...], w1_ref[...], preferred_element_type=jnp.float32)
    dis16 = jnp.broadcast_to(dis, (_BR, DH))
    h1_ref[...] = h1
    dis_ref[...] = dis16
    h1s_ref[...] = dis16 * h1


def _tc_pre(degt, x, w1):
    grid = NN // _BR
    return pl.pallas_call(
        _tc_pre_body,
        grid=(grid,),
        in_specs=[
            pl.BlockSpec((_BR, NW), lambda i: (i, 0)),
            pl.BlockSpec((_BR, DF), lambda i: (i, 0)),
            pl.BlockSpec((DF, DH), lambda i: (0, 0)),
        ],
        out_specs=[
            pl.BlockSpec((_BR, DH), lambda i: (i, 0)),
            pl.BlockSpec((_BR, DH), lambda i: (i, 0)),
            pl.BlockSpec((_BR, DH), lambda i: (i, 0)),
        ],
        out_shape=[
            jax.ShapeDtypeStruct((NN, DH), jnp.float32),
            jax.ShapeDtypeStruct((NN, DH), jnp.float32),
            jax.ShapeDtypeStruct((NN, DH), jnp.float32),
        ],
    )(degt, x, w1)


def _tc_mid_body(p_ref, h1_ref, dis_ref, b1_ref, z1_ref, z1s_ref):
    dis = dis_ref[...]
    p = p_ref[0] + p_ref[1]
    z = dis * p + dis * dis * h1_ref[...] + b1_ref[...]
    z1 = jnp.maximum(z, 0.0)
    z1_ref[...] = z1
    z1s_ref[...] = dis * z1


def _tc_mid(p, h1, dis, b1):
    grid = NN // _BR
    return pl.pallas_call(
        _tc_mid_body,
        grid=(grid,),
        in_specs=[
            pl.BlockSpec((NCORES, _BR, DH), lambda i: (0, i, 0)),
            pl.BlockSpec((_BR, DH), lambda i: (i, 0)),
            pl.BlockSpec((_BR, DH), lambda i: (i, 0)),
            pl.BlockSpec((1, DH), lambda i: (0, 0)),
        ],
        out_specs=[
            pl.BlockSpec((_BR, DH), lambda i: (i, 0)),
            pl.BlockSpec((_BR, DH), lambda i: (i, 0)),
        ],
        out_shape=[
            jax.ShapeDtypeStruct((NN, DH), jnp.float32),
            jax.ShapeDtypeStruct((NN, DH), jnp.float32),
        ],
    )(p, h1, dis, b1)


def _tc_post_body(p_ref, z1_ref, dis_ref, w2_ref, b2_ref, out_ref):
    dis = dis_ref[...]
    p = p_ref[0] + p_ref[1]
    agg = dis * p + dis * dis * z1_ref[...]
    o = jnp.dot(agg, w2_ref[...], preferred_element_type=jnp.float32)
    o = o + b2_ref[...]
    m = jnp.max(o, axis=1, keepdims=True)
    lse = jnp.log(jnp.sum(jnp.exp(o - m), axis=1, keepdims=True)) + m
    out_ref[...] = o - lse


def _tc_post(p, z1, dis, w2, b2):
    grid = NN // _BR
    return pl.pallas_call(
        _tc_post_body,
        grid=(grid,),
        in_specs=[
            pl.BlockSpec((NCORES, _BR, DH), lambda i: (0, i, 0)),
            pl.BlockSpec((_BR, DH), lambda i: (i, 0)),
            pl.BlockSpec((_BR, DH), lambda i: (i, 0)),
            pl.BlockSpec((DH, DC), lambda i: (0, 0)),
            pl.BlockSpec((1, DC), lambda i: (0, 0)),
        ],
        out_specs=pl.BlockSpec((_BR, DC), lambda i: (i, 0)),
        out_shape=jax.ShapeDtypeStruct((NN, DC), jnp.float32),
    )(p, z1, dis, w2, b2)


def kernel(x, edge_index, edge_attr, W1, b1, W2, b2):
    row = edge_index[0].astype(jnp.int32)
    col = edge_index[1].astype(jnp.int32)
    ew = edge_attr.astype(jnp.float32)
    row3 = row.reshape(NW, NCHUNK, CH)
    col3 = col.reshape(NW, NCHUNK, CH)
    ew3 = ew.reshape(NW, NCHUNK, CH)

    deg_parts = _sc_deg(col, ew)
    degt = jnp.transpose(deg_parts)
    h1, h1s, dis = _tc_pre(degt, x, W1)
    p1 = _sc_edge(h1s, row3, col3, ew3)
    z1, z1s = _tc_mid(p1, h1, dis, b1.reshape(1, DH))
    p2 = _sc_edge(z1s, row3, col3, ew3)
    out = _tc_post(p2, z1, dis, W2, b2.reshape(1, DC))
    return out


# speedup vs baseline: 53.9394x; 1.9077x over previous
import functools

import jax
import jax.numpy as jnp
from jax import lax
from jax.experimental import pallas as pl
from jax.experimental.pallas import tpu as pltpu
from jax.experimental.pallas import tpu_sc as plsc

NN = 10000
NE = 320000
DF = 128
DH = 16
DC = 40

NCORES = 2
NSUB = 16
NW = NCORES * NSUB
EPT = NE // NW
CH = 80
NCHUNK = EPT // CH
NBUF = 5
ZR = NN // NSUB

_mesh = plsc.VectorSubcoreMesh(core_axis_name="c", subcore_axis_name="s")


@functools.partial(
    pl.kernel,
    out_type=jax.ShapeDtypeStruct((NW, NN), jnp.float32),
    mesh=_mesh,
    compiler_params=pltpu.CompilerParams(needs_layout_passes=False, use_tc_tiling_on_sc=False),
    scratch_types=[
        pltpu.VMEM((EPT,), jnp.int32),
        pltpu.VMEM((EPT,), jnp.float32),
        pltpu.VMEM((NN,), jnp.float32),
    ],
)
def _sc_deg(col_hbm, ew_hbm, out_hbm, col_v, ew_v, acc_v):
    wid = lax.axis_index("s") * NCORES + lax.axis_index("c")
    base = wid * EPT
    pltpu.sync_copy(col_hbm.at[pl.ds(base, EPT)], col_v)
    pltpu.sync_copy(ew_hbm.at[pl.ds(base, EPT)], ew_v)
    zeros = jnp.zeros((16,), jnp.float32)

    def zbody(i, carry):
        acc_v[pl.ds(i * 16, 16)] = zeros
        return carry

    lax.fori_loop(0, NN // 16, zbody, 0, unroll=8)

    def body(i, carry):
        idx = col_v[pl.ds(i * 16, 16)]
        w = ew_v[pl.ds(i * 16, 16)]
        plsc.addupdate_scatter(acc_v, [idx], w)
        return carry

    lax.fori_loop(0, EPT // 16, body, 0, unroll=8)
    pltpu.sync_copy(acc_v, out_hbm.at[wid])


@functools.partial(
    pl.kernel,
    out_type=jax.ShapeDtypeStruct((NCORES, NN, DH), jnp.float32),
    mesh=_mesh,
    compiler_params=pltpu.CompilerParams(needs_layout_passes=False, use_tc_tiling_on_sc=False),
    scratch_types=[
        pltpu.VMEM((NCHUNK, CH), jnp.int32),
        pltpu.VMEM((NCHUNK, CH), jnp.int32),
        pltpu.VMEM((NCHUNK, CH), jnp.float32),
        pltpu.VMEM((NBUF, CH, DH), jnp.float32),
        pltpu.VMEM((ZR, DH), jnp.float32),
        pltpu.VMEM_SHARED((NN, DH), jnp.float32),
        [pltpu.SemaphoreType.DMA] * NBUF,
        [pltpu.SemaphoreType.DMA] * NBUF,
    ],
)
def _sc_edge(h_hbm, row_hbm, col_hbm, ew_hbm, out_hbm,
             row_v, col_v, ew_v, msg_v, zero_v, acc_sh, gsems, ssems):
    cid = lax.axis_index("c")
    sid = lax.axis_index("s")
    wid = sid * NCORES + cid
    pltpu.sync_copy(row_hbm.at[wid], row_v)
    pltpu.sync_copy(col_hbm.at[wid], col_v)
    pltpu.sync_copy(ew_hbm.at[wid], ew_v)

    zeros = jnp.zeros((16,), jnp.float32)

    def zbody(r, carry):
        zero_v[r, :] = zeros
        return carry

    lax.fori_loop(0, ZR, zbody, 0, unroll=8)
    pltpu.sync_copy(zero_v, acc_sh.at[pl.ds(sid * ZR, ZR)])
    plsc.subcore_barrier()

    def start_gather(c, b):
        pltpu.async_copy(h_hbm.at[row_v.at[c]], msg_v.at[b], gsems[b])

    def wait_gather(c, b):
        pltpu.make_async_copy(h_hbm.at[row_v.at[c]], msg_v.at[b],
                              gsems[b]).wait()

    def start_scatter(c, b):
        pltpu.async_copy(msg_v.at[b], acc_sh.at[col_v.at[c]], ssems[b],
                         add=True)

    def wait_scatter(c, b):
        pltpu.make_async_copy(msg_v.at[b], acc_sh.at[col_v.at[c]],
                              ssems[b]).wait()

    for b in range(3):
        start_gather(b, b)

    def group(g, carry):
        for b in range(NBUF):
            c = g * NBUF + b
            bn = (b + 3) % NBUF
            if b < 2:
                @pl.when(g >= 1)
                def _():
                    wait_scatter(c - 2, bn)
                start_gather(c + 3, bn)
            else:
                @pl.when(g < (NCHUNK // NBUF) - 1)
                def _():
                    wait_scatter(c - 2, bn)
                    start_gather(c + 3, bn)
            wait_gather(c, b)
            for gg in range(CH // 16):
                wvec = ew_v[c, pl.ds(gg * 16, 16)]
                for e in range(16):
                    r = gg * 16 + e
                    msg_v[b, r, :] = msg_v[b, r, :] * wvec[e]
            start_scatter(c, b)
        return carry

    lax.fori_loop(0, NCHUNK // NBUF, group, 0)
    for c in range(NCHUNK - NBUF, NCHUNK):
        wait_scatter(c, c % NBUF)
    plsc.subcore_barrier()
    pltpu.sync_copy(acc_sh.at[pl.ds(sid * ZR, ZR)],
                    out_hbm.at[cid, pl.ds(sid * ZR, ZR)])


_BR = 2000


def _tc_pre_body(degt_ref, x_ref, w1_ref, h1_ref, h1s_ref, dis_ref):
    deg = jnp.sum(degt_ref[...], axis=1, keepdims=True) + 1.0
    dis = jnp.where(deg > 0, lax.rsqrt(jnp.maximum(deg, 1e-12)), 0.0)
    h1 = jnp.dot(x_ref[...], w1_ref[...], preferred_element_type=jnp.float32)
    dis16 = jnp.broadcast_to(dis, (_BR, DH))
    h1_ref[...] = h1
    dis_ref[...] = dis16
    h1s_ref[...] = dis16 * h1


def _tc_pre(degt, x, w1):
    grid = NN // _BR
    return pl.pallas_call(
        _tc_pre_body,
        grid=(grid,),
        in_specs=[
            pl.BlockSpec((_BR, NW), lambda i: (i, 0)),
            pl.BlockSpec((_BR, DF), lambda i: (i, 0)),
            pl.BlockSpec((DF, DH), lambda i: (0, 0)),
        ],
        out_specs=[
            pl.BlockSpec((_BR, DH), lambda i: (i, 0)),
            pl.BlockSpec((_BR, DH), lambda i: (i, 0)),
            pl.BlockSpec((_BR, DH), lambda i: (i, 0)),
        ],
        out_shape=[
            jax.ShapeDtypeStruct((NN, DH), jnp.float32),
            jax.ShapeDtypeStruct((NN, DH), jnp.float32),
            jax.ShapeDtypeStruct((NN, DH), jnp.float32),
        ],
    )(degt, x, w1)


def _tc_mid_body(p_ref, h1_ref, dis_ref, b1_ref, z1_ref, z1s_ref):
    dis = dis_ref[...]
    p = p_ref[0] + p_ref[1]
    z = dis * p + dis * dis * h1_ref[...] + b1_ref[...]
    z1 = jnp.maximum(z, 0.0)
    z1_ref[...] = z1
    z1s_ref[...] = dis * z1


def _tc_mid(p, h1, dis, b1):
    grid = NN // _BR
    return pl.pallas_call(
        _tc_mid_body,
        grid=(grid,),
        in_specs=[
            pl.BlockSpec((NCORES, _BR, DH), lambda i: (0, i, 0)),
            pl.BlockSpec((_BR, DH), lambda i: (i, 0)),
            pl.BlockSpec((_BR, DH), lambda i: (i, 0)),
            pl.BlockSpec((1, DH), lambda i: (0, 0)),
        ],
        out_specs=[
            pl.BlockSpec((_BR, DH), lambda i: (i, 0)),
            pl.BlockSpec((_BR, DH), lambda i: (i, 0)),
        ],
        out_shape=[
            jax.ShapeDtypeStruct((NN, DH), jnp.float32),
            jax.ShapeDtypeStruct((NN, DH), jnp.float32),
        ],
    )(p, h1, dis, b1)


def _tc_post_body(p_ref, z1_ref, dis_ref, w2_ref, b2_ref, out_ref):
    dis = dis_ref[...]
    p = p_ref[0] + p_ref[1]
    agg = dis * p + dis * dis * z1_ref[...]
    o = jnp.dot(agg, w2_ref[...], preferred_element_type=jnp.float32)
    o = o + b2_ref[...]
    m = jnp.max(o, axis=1, keepdims=True)
    lse = jnp.log(jnp.sum(jnp.exp(o - m), axis=1, keepdims=True)) + m
    out_ref[...] = o - lse


def _tc_post(p, z1, dis, w2, b2):
    grid = NN // _BR
    return pl.pallas_call(
        _tc_post_body,
        grid=(grid,),
        in_specs=[
            pl.BlockSpec((NCORES, _BR, DH), lambda i: (0, i, 0)),
            pl.BlockSpec((_BR, DH), lambda i: (i, 0)),
            pl.BlockSpec((_BR, DH), lambda i: (i, 0)),
            pl.BlockSpec((DH, DC), lambda i: (0, 0)),
            pl.BlockSpec((1, DC), lambda i: (0, 0)),
        ],
        out_specs=pl.BlockSpec((_BR, DC), lambda i: (i, 0)),
        out_shape=jax.ShapeDtypeStruct((NN, DC), jnp.float32),
    )(p, z1, dis, w2, b2)


def kernel(x, edge_index, edge_attr, W1, b1, W2, b2):
    row = edge_index[0].astype(jnp.int32)
    col = edge_index[1].astype(jnp.int32)
    ew = edge_attr.astype(jnp.float32)
    row3 = row.reshape(NW, NCHUNK, CH)
    col3 = col.reshape(NW, NCHUNK, CH)
    ew3 = ew.reshape(NW, NCHUNK, CH)

    deg_parts = _sc_deg(col, ew)
    degt = jnp.transpose(deg_parts)
    h1, h1s, dis = _tc_pre(degt, x, W1)
    p1 = _sc_edge(h1s, row3, col3, ew3)
    z1, z1s = _tc_mid(p1, h1, dis, b1.reshape(1, DH))
    p2 = _sc_edge(z1s, row3, col3, ew3)
    out = _tc_post(p2, z1, dis, W2, b2.reshape(1, DC))
    return out
